# Initial kernel scaffold; baseline (speedup 1.0000x reference)
import jax
import jax.numpy as jnp
from jax.experimental import pallas as pl


def kernel(in_feat, edge_index, W1, b1, W2, b2, W3, b3, W4, b4):
    raise NotImplementedError("write your pallas kernel here")


# speedup vs baseline: 1.0836x; 1.0836x over previous
import functools

import jax
import jax.numpy as jnp
from jax import lax
from jax.experimental import pallas as pl
from jax.experimental.pallas import tpu as pltpu
from jax.experimental.pallas import tpu_sc as plsc

N = 10000
E = 160000
D = 256
NSC = 2
NTILE = 16
NW = NSC * NTILE
EPT = E // NW
HALF = N // NSC
CAP = 5120
GB = 64
SB = 16
BLK = 1000

@functools.cache
def _mesh():
    return plsc.VectorSubcoreMesh(core_axis_name="c", subcore_axis_name="s",
                                  num_cores=NSC, num_subcores=NTILE)


def _partition_body(src_hbm, dst_hbm, deg_part, src_list, dst_list, counts,
                    src_v, dst_v, deg_v, bs0, bd0, bs1, bd1, cnt_v):
    c = lax.axis_index("c")
    s = lax.axis_index("s")
    w = c * NTILE + s
    lanes = lax.iota(jnp.int32, 16)

    pltpu.sync_copy(src_hbm.at[pl.ds(w * EPT, EPT)], src_v.at[pl.ds(0, EPT)])
    pltpu.sync_copy(dst_hbm.at[pl.ds(w * EPT, EPT)], dst_v.at[pl.ds(0, EPT)])

    zeros16f = jnp.zeros((16,), jnp.float32)

    def zero_deg(i, carry):
        deg_v[pl.ds(i * 16, 16)] = zeros16f
        return carry

    lax.fori_loop(0, N // 16, zero_deg, 0)

    pad_src = jnp.zeros((16,), jnp.int32)
    pad_dst = jnp.full((16,), HALF, jnp.int32)

    def prefill(i, carry):
        for bs, bd in ((bs0, bd0), (bs1, bd1)):
            bs[pl.ds(i * 16, 16)] = pad_src
            bd[pl.ds(i * 16, 16)] = pad_dst
        return carry

    lax.fori_loop(0, CAP // 16, prefill, 0)

    ones16f = jnp.ones((16,), jnp.float32)

    def body(b, carry):
        c0, c1 = carry
        off = b * 16
        sv = src_v[pl.ds(off, 16)]
        dv = dst_v[pl.ds(off, 16)]
        valid = (off + lanes) < EPT
        plsc.addupdate_scatter(deg_v, [dv], ones16f, mask=valid)
        cs = []
        for k, ck, bs, bd in ((0, c0, bs0, bd0), (1, c1, bs1, bd1)):
            if k == 0:
                m = (dv < HALF) & valid
            else:
                m = (dv >= HALF) & valid
            plsc.store_compressed(bs.at[pl.ds(ck, 16)], sv, mask=m)
            plsc.store_compressed(bd.at[pl.ds(ck, 16)], dv - k * HALF, mask=m)
            cs.append(ck + jnp.sum(m.astype(jnp.int32)))
        return tuple(cs)

    nb = (EPT + 15) // 16
    c0, c1 = lax.fori_loop(0, nb, body, (jnp.int32(0), jnp.int32(0)))

    bs0[pl.ds(c0, 16)] = pad_src
    bd0[pl.ds(c0, 16)] = pad_dst
    bs1[pl.ds(c1, 16)] = pad_src
    bd1[pl.ds(c1, 16)] = pad_dst

    cnt = jnp.where(lanes == 0, c0, jnp.where(lanes == 1, c1, 0)).astype(jnp.int32)
    cnt_v[...] = cnt
    pltpu.sync_copy(cnt_v, counts.at[w])
    pltpu.sync_copy(deg_v, deg_part.at[w])
    pltpu.sync_copy(bs0, src_list.at[w, 0])
    pltpu.sync_copy(bs1, src_list.at[w, 1])
    pltpu.sync_copy(bd0, dst_list.at[w, 0])
    pltpu.sync_copy(bd1, dst_list.at[w, 1])


@functools.cache
def _partition_kernel():
    return pl.kernel(
        _partition_body,
        out_type=(
            jax.ShapeDtypeStruct((NW, N), jnp.float32),
            jax.ShapeDtypeStruct((NW, 2, CAP), jnp.int32),
            jax.ShapeDtypeStruct((NW, 2, CAP), jnp.int32),
            jax.ShapeDtypeStruct((NW, 16), jnp.int32),
        ),
        mesh=_mesh(),
        compiler_params=pltpu.CompilerParams(needs_layout_passes=False),
        scratch_types=[
            pltpu.VMEM((EPT + 16,), jnp.int32),
            pltpu.VMEM((EPT + 16,), jnp.int32),
            pltpu.VMEM((N,), jnp.float32),
            pltpu.VMEM((CAP,), jnp.int32),
            pltpu.VMEM((CAP,), jnp.int32),
            pltpu.VMEM((CAP,), jnp.int32),
            pltpu.VMEM((CAP,), jnp.int32),
            pltpu.VMEM((16,), jnp.int32),
        ],
    )


def _partition(src, dst):
    return _partition_kernel()(src, dst)


EB = 128
RNG = N // NTILE
ACC_ROWS = RNG + 7
CHW = D // NSC


def _flush(u_hbm, stg_src, stg_dst, rb, acc, sem, col0, lanes):
    pltpu.async_copy(
        u_hbm.at[stg_src.at[pl.ds(0, EB)], pl.ds(col0, CHW)], rb, sem).wait()
    for i in range(EB // 16):
        dv = stg_dst[pl.ds(i * 16, 16)]
        eix = i * 16 + lanes
        for c in range(CHW):
            cv16 = jnp.full((16,), c, jnp.int32)
            vals = plsc.load_gather(rb, [eix, cv16])
            plsc.addupdate_scatter(acc, [dv, cv16], vals)


def _prop_body(u_hbm, src_list, dst_list, counts, out_hbm,
               ss, sd, stg_src, stg_dst, cball, acc, rb, sem):
    sc = lax.axis_index("c")
    t = lax.axis_index("s")
    lanes = lax.iota(jnp.int32, 16)
    col0 = sc * CHW
    k = t // 8
    lo = t * RNG - k * HALF
    hi = lo + RNG

    zeros16f = jnp.zeros((16,), jnp.float32)

    def zacc(i, carry):
        for c in range(CHW // 16):
            acc[i, pl.ds(c * 16, 16)] = zeros16f
        return carry

    lax.fori_loop(0, ACC_ROWS, zacc, 0)

    pltpu.sync_copy(counts, cball)

    pad_src = jnp.zeros((16,), jnp.int32)
    pad_dst = jnp.full((16,), RNG, jnp.int32)

    def seg(w, cnt_s):
        cv = cball[w, pl.ds(0, 16)]
        cnt = jnp.max(jnp.where(lanes == k, cv, 0))
        nch = (cnt + 1023) // 1024

        def load_chunk(cc, carry):
            pltpu.sync_copy(src_list.at[w, k, pl.ds(cc * 1024, 1024)],
                            ss.at[pl.ds(cc * 1024, 1024)])
            pltpu.sync_copy(dst_list.at[w, k, pl.ds(cc * 1024, 1024)],
                            sd.at[pl.ds(cc * 1024, 1024)])
            return carry

        lax.fori_loop(0, nch, load_chunk, 0)

        nv = (cnt + 15) // 16

        def scan(j, cs):
            sv = ss[pl.ds(j * 16, 16)]
            dv = sd[pl.ds(j * 16, 16)]
            m = (dv >= lo) & (dv < hi) & ((j * 16 + lanes) < cnt)
            plsc.store_compressed(stg_src.at[pl.ds(cs, 16)], sv, mask=m)
            plsc.store_compressed(stg_dst.at[pl.ds(cs, 16)], dv - lo, mask=m)
            cs = cs + jnp.sum(m.astype(jnp.int32))

            @pl.when(cs >= EB)
            def _():
                _flush(u_hbm, stg_src, stg_dst, rb, acc, sem, col0, lanes)
                stg_src[pl.ds(0, 16)] = stg_src[pl.ds(EB, 16)]
                stg_dst[pl.ds(0, 16)] = stg_dst[pl.ds(EB, 16)]

            return jnp.where(cs >= EB, cs - EB, cs)

        return lax.fori_loop(0, nv, scan, cnt_s)

    cnt_s = lax.fori_loop(0, NW, seg, jnp.int32(0))

    @pl.when(cnt_s > 0)
    def _():
        for q in range(EB // 16):
            pos = cnt_s + q * 16
            stg_src[pl.ds(pos, 16)] = pad_src
            stg_dst[pl.ds(pos, 16)] = pad_dst
        _flush(u_hbm, stg_src, stg_dst, rb, acc, sem, col0, lanes)

    pltpu.sync_copy(acc.at[pl.ds(0, RNG)],
                    out_hbm.at[t, pl.ds(0, RNG), pl.ds(col0, CHW)])


@functools.cache
def _prop_kernel():
    return pl.kernel(
        _prop_body,
        out_type=jax.ShapeDtypeStruct((NTILE, RNG, D), jnp.float32),
        mesh=_mesh(),
        compiler_params=pltpu.CompilerParams(needs_layout_passes=False),
        scratch_types=[
            pltpu.VMEM((CAP,), jnp.int32),
            pltpu.VMEM((CAP,), jnp.int32),
            pltpu.VMEM((2 * EB,), jnp.int32),
            pltpu.VMEM((2 * EB,), jnp.int32),
            pltpu.VMEM((NW, 16), jnp.int32),
            pltpu.VMEM((ACC_ROWS, CHW), jnp.float32),
            pltpu.VMEM((EB, CHW), jnp.float32),
            pltpu.SemaphoreType.DMA,
        ],
    )


def _prop(u, src_list, dst_list, counts):
    out3 = _prop_kernel()(u, src_list, dst_list, counts)
    return out3.reshape(N, D)


def _dotT(a, w):
    return lax.dot_general(a, w, (((1,), (1,)), ((), ())),
                           preferred_element_type=jnp.float32)


def _dinv_body(deg_ref, dinv_ref):
    deg = jnp.sum(deg_ref[...], axis=0)
    dinv_ref[...] = jnp.where(deg > 0, 1.0 / jnp.sqrt(deg), 0.0)[:, None]


def _dinv(deg_part):
    return pl.pallas_call(
        _dinv_body,
        out_shape=jax.ShapeDtypeStruct((N, 1), jnp.float32),
    )(deg_part)


def _mlp_body(x_ref, w1_ref, b1_ref, w2_ref, b2_ref, dinv_ref,
              h_ref, u_ref):
    x = x_ref[...]
    h1 = jnp.maximum(_dotT(x, w1_ref[...]) + b1_ref[...], 0.0)
    h2 = jnp.maximum(_dotT(h1, w2_ref[...]) + b2_ref[...], 0.0)
    h_ref[...] = h2
    u_ref[...] = h2 * dinv_ref[...]


def _mlp(x, W1, b1, W2, b2, dinv):
    return pl.pallas_call(
        _mlp_body,
        grid=(N // BLK,),
        in_specs=[
            pl.BlockSpec((BLK, D), lambda i: (i, 0)),
            pl.BlockSpec((D, D), lambda i: (0, 0)),
            pl.BlockSpec((D,), lambda i: (0,)),
            pl.BlockSpec((D, D), lambda i: (0, 0)),
            pl.BlockSpec((D,), lambda i: (0,)),
            pl.BlockSpec((BLK, 1), lambda i: (i, 0)),
        ],
        out_specs=[
            pl.BlockSpec((BLK, D), lambda i: (i, 0)),
            pl.BlockSpec((BLK, D), lambda i: (i, 0)),
        ],
        out_shape=[
            jax.ShapeDtypeStruct((N, D), jnp.float32),
            jax.ShapeDtypeStruct((N, D), jnp.float32),
        ],
    )(x, W1, b1, W2, b2, dinv)


def _p1_body(h_ref, s_ref, dinv_ref, p1_ref, u1_ref):
    dinv = dinv_ref[...]
    p1 = h_ref[...] - dinv * s_ref[...]
    p1_ref[...] = p1
    u1_ref[...] = dinv * p1


def _p1(h, S0, dinv):
    return pl.pallas_call(
        _p1_body,
        grid=(N // BLK,),
        in_specs=[
            pl.BlockSpec((BLK, D), lambda i: (i, 0)),
            pl.BlockSpec((BLK, D), lambda i: (i, 0)),
            pl.BlockSpec((BLK, 1), lambda i: (i, 0)),
        ],
        out_specs=[
            pl.BlockSpec((BLK, D), lambda i: (i, 0)),
            pl.BlockSpec((BLK, D), lambda i: (i, 0)),
        ],
        out_shape=[
            jax.ShapeDtypeStruct((N, D), jnp.float32),
            jax.ShapeDtypeStruct((N, D), jnp.float32),
        ],
    )(h, S0, dinv)


def _final_body(h_ref, p1_ref, s1_ref, dinv_ref, v0_ref, v1_ref, v2_ref,
                b3_ref, w4_ref, b4_ref, o_ref):
    dinv = dinv_ref[...]
    p1 = p1_ref[...]
    p2 = p1 - dinv * s1_ref[...]
    z = (_dotT(h_ref[...], v0_ref[...]) + _dotT(p1, v1_ref[...])
         + _dotT(p2, v2_ref[...]) + b3_ref[...])
    z = jnp.maximum(z, 0.0)
    o_ref[...] = _dotT(z, w4_ref[...]) + b4_ref[...]


def _final(h, p1, S1, dinv, V0, V1, V2, b3, W4, b4):
    return pl.pallas_call(
        _final_body,
        grid=(N // BLK,),
        in_specs=[
            pl.BlockSpec((BLK, D), lambda i: (i, 0)),
            pl.BlockSpec((BLK, D), lambda i: (i, 0)),
            pl.BlockSpec((BLK, D), lambda i: (i, 0)),
            pl.BlockSpec((BLK, 1), lambda i: (i, 0)),
            pl.BlockSpec((D, D), lambda i: (0, 0)),
            pl.BlockSpec((D, D), lambda i: (0, 0)),
            pl.BlockSpec((D, D), lambda i: (0, 0)),
            pl.BlockSpec((D,), lambda i: (0,)),
            pl.BlockSpec((2, D), lambda i: (0, 0)),
            pl.BlockSpec((2,), lambda i: (0,)),
        ],
        out_specs=pl.BlockSpec((BLK, 2), lambda i: (i, 0)),
        out_shape=jax.ShapeDtypeStruct((N, 2), jnp.float32),
    )(h, p1, S1, dinv, V0, V1, V2, b3, W4, b4)


def kernel(in_feat, edge_index, W1, b1, W2, b2, W3, b3, W4, b4):
    edges = edge_index.astype(jnp.int32)
    deg_part, src_list, dst_list, counts = _partition(edges[0], edges[1])
    dinv = _dinv(deg_part)
    h, u0 = _mlp(in_feat, W1, b1, W2, b2, dinv)
    S0 = _prop(u0, src_list, dst_list, counts)
    p1, u1 = _p1(h, S0, dinv)
    S1 = _prop(u1, src_list, dst_list, counts)
    W3a = W3[:, :D]
    W3b = W3[:, D:2 * D]
    W3c = W3[:, 2 * D:]
    V0 = 3.0 * W3a
    V1 = 3.0 * (W3b - W3a)
    V2 = 0.75 * (W3a - 2.0 * W3b + W3c)
    return _final(h, p1, S1, dinv, V0, V1, V2, b3, W4, b4)


# speedup vs baseline: 2.9801x; 2.7501x over previous
import functools

import jax
import jax.numpy as jnp
from jax import lax
from jax.experimental import pallas as pl
from jax.experimental.pallas import tpu as pltpu
from jax.experimental.pallas import tpu_sc as plsc

N = 10000
E = 160000
D = 256
NSC = 2
NTILE = 16
NW = NSC * NTILE
EPT = E // NW
HALF = N // NSC
CAP = 5120
GB = 64
SB = 16
BLK = 1000

@functools.cache
def _mesh():
    return plsc.VectorSubcoreMesh(core_axis_name="c", subcore_axis_name="s",
                                  num_cores=NSC, num_subcores=NTILE)


def _partition_body(src_hbm, dst_hbm, deg_part, src_list, dst_list, counts,
                    src_v, dst_v, deg_v, bs0, bd0, bs1, bd1, cnt_v):
    c = lax.axis_index("c")
    s = lax.axis_index("s")
    w = c * NTILE + s
    lanes = lax.iota(jnp.int32, 16)

    pltpu.sync_copy(src_hbm.at[pl.ds(w * EPT, EPT)], src_v.at[pl.ds(0, EPT)])
    pltpu.sync_copy(dst_hbm.at[pl.ds(w * EPT, EPT)], dst_v.at[pl.ds(0, EPT)])

    zeros16f = jnp.zeros((16,), jnp.float32)

    def zero_deg(i, carry):
        deg_v[pl.ds(i * 16, 16)] = zeros16f
        return carry

    lax.fori_loop(0, N // 16, zero_deg, 0)

    pad_src = jnp.zeros((16,), jnp.int32)
    pad_dst = jnp.full((16,), HALF, jnp.int32)

    def prefill(i, carry):
        for bs, bd in ((bs0, bd0), (bs1, bd1)):
            bs[pl.ds(i * 16, 16)] = pad_src
            bd[pl.ds(i * 16, 16)] = pad_dst
        return carry

    lax.fori_loop(0, CAP // 16, prefill, 0)

    ones16f = jnp.ones((16,), jnp.float32)

    def body(b, carry):
        c0, c1 = carry
        off = b * 16
        sv = src_v[pl.ds(off, 16)]
        dv = dst_v[pl.ds(off, 16)]
        valid = (off + lanes) < EPT
        plsc.addupdate_scatter(deg_v, [dv], ones16f, mask=valid)
        cs = []
        for k, ck, bs, bd in ((0, c0, bs0, bd0), (1, c1, bs1, bd1)):
            if k == 0:
                m = (dv < HALF) & valid
            else:
                m = (dv >= HALF) & valid
            plsc.store_compressed(bs.at[pl.ds(ck, 16)], sv, mask=m)
            plsc.store_compressed(bd.at[pl.ds(ck, 16)], dv - k * HALF, mask=m)
            cs.append(ck + jnp.sum(m.astype(jnp.int32)))
        return tuple(cs)

    nb = (EPT + 15) // 16
    c0, c1 = lax.fori_loop(0, nb, body, (jnp.int32(0), jnp.int32(0)))

    bs0[pl.ds(c0, 16)] = pad_src
    bd0[pl.ds(c0, 16)] = pad_dst
    bs1[pl.ds(c1, 16)] = pad_src
    bd1[pl.ds(c1, 16)] = pad_dst

    cnt = jnp.where(lanes == 0, c0, jnp.where(lanes == 1, c1, 0)).astype(jnp.int32)
    cnt_v[...] = cnt
    pltpu.sync_copy(cnt_v, counts.at[w])
    pltpu.sync_copy(deg_v, deg_part.at[w])
    pltpu.sync_copy(bs0, src_list.at[w, 0])
    pltpu.sync_copy(bs1, src_list.at[w, 1])
    pltpu.sync_copy(bd0, dst_list.at[w, 0])
    pltpu.sync_copy(bd1, dst_list.at[w, 1])


@functools.cache
def _partition_kernel():
    return pl.kernel(
        _partition_body,
        out_type=(
            jax.ShapeDtypeStruct((NW, N), jnp.float32),
            jax.ShapeDtypeStruct((NW, 2, CAP), jnp.int32),
            jax.ShapeDtypeStruct((NW, 2, CAP), jnp.int32),
            jax.ShapeDtypeStruct((NW, 16), jnp.int32),
        ),
        mesh=_mesh(),
        compiler_params=pltpu.CompilerParams(needs_layout_passes=False),
        scratch_types=[
            pltpu.VMEM((EPT + 16,), jnp.int32),
            pltpu.VMEM((EPT + 16,), jnp.int32),
            pltpu.VMEM((N,), jnp.float32),
            pltpu.VMEM((CAP,), jnp.int32),
            pltpu.VMEM((CAP,), jnp.int32),
            pltpu.VMEM((CAP,), jnp.int32),
            pltpu.VMEM((CAP,), jnp.int32),
            pltpu.VMEM((16,), jnp.int32),
        ],
    )


def _partition(src, dst):
    return _partition_kernel()(src, dst)


EB = 128
RNG = N // NTILE
ACC_ROWS = RNG + 7
CHW = D // NSC


def _flush(u_hbm, stg_src, stg_dst, rb, acc, sem, col0, lanes):
    pltpu.async_copy(
        u_hbm.at[stg_src.at[pl.ds(0, EB)], pl.ds(col0, CHW)], rb, sem).wait()

    def edges8(i, carry):
        for e in range(8):
            eidx = jnp.full((16,), i * 8 + e, jnp.int32)
            dvb = plsc.load_gather(stg_dst, [eidx])
            for c in range(CHW // 16):
                vals = rb[i * 8 + e, pl.ds(c * 16, 16)]
                plsc.addupdate_scatter(acc, [dvb, c * 16 + lanes], vals)
        return carry

    lax.fori_loop(0, EB // 8, edges8, 0)


def _prop_body(u_hbm, src_list, dst_list, counts, out_hbm,
               ss, sd, stg_src, stg_dst, cball, acc, rb, sem):
    sc = lax.axis_index("c")
    t = lax.axis_index("s")
    lanes = lax.iota(jnp.int32, 16)
    col0 = sc * CHW
    k = t // 8
    lo = t * RNG - k * HALF
    hi = lo + RNG

    zeros16f = jnp.zeros((16,), jnp.float32)

    def zacc(i, carry):
        for c in range(CHW // 16):
            acc[i, pl.ds(c * 16, 16)] = zeros16f
        return carry

    lax.fori_loop(0, ACC_ROWS, zacc, 0)

    pltpu.sync_copy(counts, cball)

    pad_src = jnp.zeros((16,), jnp.int32)
    pad_dst = jnp.full((16,), RNG, jnp.int32)

    def seg(w, cnt_s):
        cv = cball[w, pl.ds(0, 16)]
        cnt = jnp.max(jnp.where(lanes == k, cv, 0))
        nch = (cnt + 1023) // 1024

        def load_chunk(cc, carry):
            pltpu.sync_copy(src_list.at[w, k, pl.ds(cc * 1024, 1024)],
                            ss.at[pl.ds(cc * 1024, 1024)])
            pltpu.sync_copy(dst_list.at[w, k, pl.ds(cc * 1024, 1024)],
                            sd.at[pl.ds(cc * 1024, 1024)])
            return carry

        lax.fori_loop(0, nch, load_chunk, 0)

        nv = (cnt + 15) // 16

        def scan(j, cs):
            sv = ss[pl.ds(j * 16, 16)]
            dv = sd[pl.ds(j * 16, 16)]
            m = (dv >= lo) & (dv < hi) & ((j * 16 + lanes) < cnt)
            plsc.store_compressed(stg_src.at[pl.ds(cs, 16)], sv, mask=m)
            plsc.store_compressed(stg_dst.at[pl.ds(cs, 16)], dv - lo, mask=m)
            cs = cs + jnp.sum(m.astype(jnp.int32))

            @pl.when(cs >= EB)
            def _():
                _flush(u_hbm, stg_src, stg_dst, rb, acc, sem, col0, lanes)
                stg_src[pl.ds(0, 16)] = stg_src[pl.ds(EB, 16)]
                stg_dst[pl.ds(0, 16)] = stg_dst[pl.ds(EB, 16)]

            return jnp.where(cs >= EB, cs - EB, cs)

        return lax.fori_loop(0, nv, scan, cnt_s)

    cnt_s = lax.fori_loop(0, NW, seg, jnp.int32(0))

    @pl.when(cnt_s > 0)
    def _():
        for q in range(EB // 16):
            pos = cnt_s + q * 16
            stg_src[pl.ds(pos, 16)] = pad_src
            stg_dst[pl.ds(pos, 16)] = pad_dst
        _flush(u_hbm, stg_src, stg_dst, rb, acc, sem, col0, lanes)

    pltpu.sync_copy(acc.at[pl.ds(0, RNG)],
                    out_hbm.at[t, pl.ds(0, RNG), pl.ds(col0, CHW)])


@functools.cache
def _prop_kernel():
    return pl.kernel(
        _prop_body,
        out_type=jax.ShapeDtypeStruct((NTILE, RNG, D), jnp.float32),
        mesh=_mesh(),
        compiler_params=pltpu.CompilerParams(needs_layout_passes=False),
        scratch_types=[
            pltpu.VMEM((CAP,), jnp.int32),
            pltpu.VMEM((CAP,), jnp.int32),
            pltpu.VMEM((2 * EB,), jnp.int32),
            pltpu.VMEM((2 * EB,), jnp.int32),
            pltpu.VMEM((NW, 16), jnp.int32),
            pltpu.VMEM((ACC_ROWS, CHW), jnp.float32),
            pltpu.VMEM((EB, CHW), jnp.float32),
            pltpu.SemaphoreType.DMA,
        ],
    )


def _prop(u, src_list, dst_list, counts):
    out3 = _prop_kernel()(u, src_list, dst_list, counts)
    return out3.reshape(N, D)


def _dotT(a, w):
    return lax.dot_general(a, w, (((1,), (1,)), ((), ())),
                           preferred_element_type=jnp.float32)


def _dinv_body(deg_ref, dinv_ref):
    deg = jnp.sum(deg_ref[...], axis=0)
    dinv_ref[...] = jnp.where(deg > 0, 1.0 / jnp.sqrt(deg), 0.0)[:, None]


def _dinv(deg_part):
    return pl.pallas_call(
        _dinv_body,
        out_shape=jax.ShapeDtypeStruct((N, 1), jnp.float32),
    )(deg_part)


def _mlp_body(x_ref, w1_ref, b1_ref, w2_ref, b2_ref, dinv_ref,
              h_ref, u_ref):
    x = x_ref[...]
    h1 = jnp.maximum(_dotT(x, w1_ref[...]) + b1_ref[...], 0.0)
    h2 = jnp.maximum(_dotT(h1, w2_ref[...]) + b2_ref[...], 0.0)
    h_ref[...] = h2
    u_ref[...] = h2 * dinv_ref[...]


def _mlp(x, W1, b1, W2, b2, dinv):
    return pl.pallas_call(
        _mlp_body,
        grid=(N // BLK,),
        in_specs=[
            pl.BlockSpec((BLK, D), lambda i: (i, 0)),
            pl.BlockSpec((D, D), lambda i: (0, 0)),
            pl.BlockSpec((D,), lambda i: (0,)),
            pl.BlockSpec((D, D), lambda i: (0, 0)),
            pl.BlockSpec((D,), lambda i: (0,)),
            pl.BlockSpec((BLK, 1), lambda i: (i, 0)),
        ],
        out_specs=[
            pl.BlockSpec((BLK, D), lambda i: (i, 0)),
            pl.BlockSpec((BLK, D), lambda i: (i, 0)),
        ],
        out_shape=[
            jax.ShapeDtypeStruct((N, D), jnp.float32),
            jax.ShapeDtypeStruct((N, D), jnp.float32),
        ],
    )(x, W1, b1, W2, b2, dinv)


def _p1_body(h_ref, s_ref, dinv_ref, p1_ref, u1_ref):
    dinv = dinv_ref[...]
    p1 = h_ref[...] - dinv * s_ref[...]
    p1_ref[...] = p1
    u1_ref[...] = dinv * p1


def _p1(h, S0, dinv):
    return pl.pallas_call(
        _p1_body,
        grid=(N // BLK,),
        in_specs=[
            pl.BlockSpec((BLK, D), lambda i: (i, 0)),
            pl.BlockSpec((BLK, D), lambda i: (i, 0)),
            pl.BlockSpec((BLK, 1), lambda i: (i, 0)),
        ],
        out_specs=[
            pl.BlockSpec((BLK, D), lambda i: (i, 0)),
            pl.BlockSpec((BLK, D), lambda i: (i, 0)),
        ],
        out_shape=[
            jax.ShapeDtypeStruct((N, D), jnp.float32),
            jax.ShapeDtypeStruct((N, D), jnp.float32),
        ],
    )(h, S0, dinv)


def _final_body(h_ref, p1_ref, s1_ref, dinv_ref, v0_ref, v1_ref, v2_ref,
                b3_ref, w4_ref, b4_ref, o_ref):
    dinv = dinv_ref[...]
    p1 = p1_ref[...]
    p2 = p1 - dinv * s1_ref[...]
    z = (_dotT(h_ref[...], v0_ref[...]) + _dotT(p1, v1_ref[...])
         + _dotT(p2, v2_ref[...]) + b3_ref[...])
    z = jnp.maximum(z, 0.0)
    o_ref[...] = _dotT(z, w4_ref[...]) + b4_ref[...]


def _final(h, p1, S1, dinv, V0, V1, V2, b3, W4, b4):
    return pl.pallas_call(
        _final_body,
        grid=(N // BLK,),
        in_specs=[
            pl.BlockSpec((BLK, D), lambda i: (i, 0)),
            pl.BlockSpec((BLK, D), lambda i: (i, 0)),
            pl.BlockSpec((BLK, D), lambda i: (i, 0)),
            pl.BlockSpec((BLK, 1), lambda i: (i, 0)),
            pl.BlockSpec((D, D), lambda i: (0, 0)),
            pl.BlockSpec((D, D), lambda i: (0, 0)),
            pl.BlockSpec((D, D), lambda i: (0, 0)),
            pl.BlockSpec((D,), lambda i: (0,)),
            pl.BlockSpec((2, D), lambda i: (0, 0)),
            pl.BlockSpec((2,), lambda i: (0,)),
        ],
        out_specs=pl.BlockSpec((BLK, 2), lambda i: (i, 0)),
        out_shape=jax.ShapeDtypeStruct((N, 2), jnp.float32),
    )(h, p1, S1, dinv, V0, V1, V2, b3, W4, b4)


def kernel(in_feat, edge_index, W1, b1, W2, b2, W3, b3, W4, b4):
    edges = edge_index.astype(jnp.int32)
    deg_part, src_list, dst_list, counts = _partition(edges[0], edges[1])
    dinv = _dinv(deg_part)
    h, u0 = _mlp(in_feat, W1, b1, W2, b2, dinv)
    S0 = _prop(u0, src_list, dst_list, counts)
    p1, u1 = _p1(h, S0, dinv)
    S1 = _prop(u1, src_list, dst_list, counts)
    W3a = W3[:, :D]
    W3b = W3[:, D:2 * D]
    W3c = W3[:, 2 * D:]
    V0 = 3.0 * W3a
    V1 = 3.0 * (W3b - W3a)
    V2 = 0.75 * (W3a - 2.0 * W3b + W3c)
    return _final(h, p1, S1, dinv, V0, V1, V2, b3, W4, b4)
